# baseline (speedup 1.0000x reference)
import functools

import jax
import jax.numpy as jnp
from jax import lax
from jax.experimental import pallas as pl
from jax.experimental.pallas import tpu as pltpu

DIM = 32
K = 8192
N = 4096
BLK = 256
GRID = N // BLK


CH = 1024
NCH = K // CH


def _vq_body(x_ref, w_ref, d_ref, idx_ref, enc_ref, q_ref, pplx_ref, counts):
    i = pl.program_id(0)
    x = x_ref[...]
    w = w_ref[...]
    x2 = jnp.sum(x * x, axis=1, keepdims=True)
    w2 = jnp.sum(w * w, axis=1)[None, :]
    mm2 = lax.dot_general(x, w + w, (((1,), (1,)), ((), ())),
                          preferred_element_type=jnp.float32)

    run_min = None
    run_idx = None
    for c in range(NCH):
        sl = slice(c * CH, (c + 1) * CH)
        d = (x2 + w2[:, sl]) - mm2[:, sl]
        d_ref[:, sl] = d
        col = lax.broadcasted_iota(jnp.int32, (BLK, CH), 1) + (c * CH)
        dmin = jnp.min(d, axis=1, keepdims=True)
        cidx = jnp.min(jnp.where(d == dmin, col, K), axis=1, keepdims=True)
        if c == 0:
            run_min, run_idx = dmin, cidx
        else:
            better = dmin < run_min
            run_idx = jnp.where(better, cidx, run_idx)
            run_min = jnp.where(better, dmin, run_min)

    idx_ref[...] = run_idx

    col = lax.broadcasted_iota(jnp.int32, (BLK, K), 1)
    onehot = (col == run_idx).astype(jnp.float32)
    enc_ref[...] = onehot
    q = lax.dot_general(onehot, w, (((1,), (0,)), ((), ())),
                        preferred_element_type=jnp.float32)
    q_ref[...] = x + (q - x)

    ones = jnp.ones((1, BLK), jnp.float32)
    cnt = lax.dot_general(ones, onehot, (((1,), (0,)), ((), ())),
                          preferred_element_type=jnp.float32)

    @pl.when(i == 0)
    def _init():
        counts[...] = cnt

    @pl.when(i > 0)
    def _acc():
        counts[...] += cnt

    @pl.when(i == GRID - 1)
    def _finish():
        avg = counts[...] * (1.0 / N)
        s = jnp.sum(avg * jnp.log(avg + 1e-10))
        pplx_ref[...] = jnp.exp(-s).reshape(1, 1)


@jax.jit
def kernel(inputs, weight):
    x = jnp.transpose(inputs, (0, 2, 3, 1))
    input_shape = x.shape
    flat = x.reshape(-1, DIM)

    d, idx, enc, q, pplx = pl.pallas_call(
        _vq_body,
        grid=(GRID,),
        in_specs=[
            pl.BlockSpec((BLK, DIM), lambda i: (i, 0)),
            pl.BlockSpec((K, DIM), lambda i: (0, 0)),
        ],
        out_specs=[
            pl.BlockSpec((BLK, K), lambda i: (i, 0)),
            pl.BlockSpec((BLK, 1), lambda i: (i, 0)),
            pl.BlockSpec((BLK, K), lambda i: (i, 0)),
            pl.BlockSpec((BLK, DIM), lambda i: (i, 0)),
            pl.BlockSpec((1, 1), lambda i: (0, 0)),
        ],
        out_shape=[
            jax.ShapeDtypeStruct((N, K), jnp.float32),
            jax.ShapeDtypeStruct((N, 1), jnp.int32),
            jax.ShapeDtypeStruct((N, K), jnp.float32),
            jax.ShapeDtypeStruct((N, DIM), jnp.float32),
            jax.ShapeDtypeStruct((1, 1), jnp.float32),
        ],
        scratch_shapes=[pltpu.VMEM((1, K), jnp.float32)],
    )(flat, weight)

    quantized = jnp.transpose(q.reshape(input_shape), (0, 3, 1, 2))
    return (d, enc, idx, quantized, pplx.reshape(()))


# speedup vs baseline: 1.0068x; 1.0068x over previous
import functools

import jax
import jax.numpy as jnp
from jax import lax
from jax.experimental import pallas as pl
from jax.experimental.pallas import tpu as pltpu

DIM = 32
K = 8192
N = 4096
BLK = 256
GRID = N // BLK


def _vq_body(x_ref, w_ref, d_ref, idx_ref, enc_ref, q_ref, pplx_ref, counts):
    i = pl.program_id(0)
    x = x_ref[...]
    w = w_ref[...]
    x2 = jnp.sum(x * x, axis=1, keepdims=True)
    w2 = jnp.sum(w * w, axis=1)
    mm2 = lax.dot_general(x, w + w, (((1,), (1,)), ((), ())),
                          preferred_element_type=jnp.float32)
    d = (x2 + w2[None, :]) - mm2
    d_ref[...] = d

    col = lax.broadcasted_iota(jnp.int32, (BLK, K), 1)
    dmin = jnp.min(d, axis=1, keepdims=True)
    idx = jnp.min(jnp.where(d == dmin, col, K), axis=1)
    idx_ref[...] = idx[:, None]

    onehot = (col == idx[:, None]).astype(jnp.float32)
    enc_ref[...] = onehot
    q = lax.dot_general(onehot, w, (((1,), (0,)), ((), ())),
                        preferred_element_type=jnp.float32)
    q_ref[...] = x + (q - x)

    ones = jnp.ones((1, BLK), jnp.float32)
    cnt = lax.dot_general(ones, onehot, (((1,), (0,)), ((), ())),
                          preferred_element_type=jnp.float32)

    @pl.when(i == 0)
    def _init():
        counts[...] = cnt

    @pl.when(i > 0)
    def _acc():
        counts[...] += cnt

    @pl.when(i == GRID - 1)
    def _finish():
        avg = counts[...] * (1.0 / N)
        s = jnp.sum(avg * jnp.log(avg + 1e-10))
        pplx_ref[...] = jnp.exp(-s).reshape(1, 1)


@jax.jit
def kernel(inputs, weight):
    x = jnp.transpose(inputs, (0, 2, 3, 1))
    input_shape = x.shape
    flat = x.reshape(-1, DIM)

    d, idx, enc, q, pplx = pl.pallas_call(
        _vq_body,
        grid=(GRID,),
        in_specs=[
            pl.BlockSpec((BLK, DIM), lambda i: (i, 0)),
            pl.BlockSpec((K, DIM), lambda i: (0, 0)),
        ],
        out_specs=[
            pl.BlockSpec((BLK, K), lambda i: (i, 0)),
            pl.BlockSpec((BLK, 1), lambda i: (i, 0)),
            pl.BlockSpec((BLK, K), lambda i: (i, 0)),
            pl.BlockSpec((BLK, DIM), lambda i: (i, 0)),
            pl.BlockSpec((1, 1), lambda i: (0, 0)),
        ],
        out_shape=[
            jax.ShapeDtypeStruct((N, K), jnp.float32),
            jax.ShapeDtypeStruct((N, 1), jnp.int32),
            jax.ShapeDtypeStruct((N, K), jnp.float32),
            jax.ShapeDtypeStruct((N, DIM), jnp.float32),
            jax.ShapeDtypeStruct((1, 1), jnp.float32),
        ],
        scratch_shapes=[pltpu.VMEM((1, K), jnp.float32)],
    )(flat, weight)

    quantized = jnp.transpose(q.reshape(input_shape), (0, 3, 1, 2))
    return (d, enc, idx, quantized, pplx.reshape(()))
